# Initial kernel scaffold; baseline (speedup 1.0000x reference)
import jax
import jax.numpy as jnp
from jax.experimental import pallas as pl


def kernel(feat, shuf_feat, edge_index, dif_edge_index, W1, b1, W2, b2, Bw, Bb):
    raise NotImplementedError("write your pallas kernel here")


# speedup vs baseline: 21.5291x; 21.5291x over previous
import functools

import jax
import jax.numpy as jnp
from jax import lax
from jax.experimental import pallas as pl
from jax.experimental.pallas import tpu as pltpu
from jax.experimental.pallas import tpu_sc as plsc

N = 10000
E = 320000
D = 128

NC = 2
NS = 16
NW = NC * NS
LANE = 128
R = -(-E // (NW * LANE))
CHUNK = R * LANE
E_PAD = NW * CHUNK
N_PAD = 10240

_mesh = plsc.VectorSubcoreMesh(core_axis_name="c", subcore_axis_name="s")


def _bincount_body(src1, src2, ones_h, zeros_h, out, idx_v, val_v, acc1, acc2):
    c = lax.axis_index("c")
    s = lax.axis_index("s")
    tid = c * NS + s

    @pl.when(s == 0)
    def _zero():
        pltpu.sync_copy(zeros_h, acc1)
        pltpu.sync_copy(zeros_h, acc2)

    plsc.subcore_barrier()
    pltpu.sync_copy(ones_h, val_v)

    pltpu.sync_copy(src1.at[tid], idx_v)

    def _scat1(j, carry):
        pltpu.sync_copy(val_v.at[j], acc1.at[idx_v.at[j]], add=True)
        return carry

    lax.fori_loop(0, R, _scat1, 0)
    pltpu.sync_copy(src2.at[tid], idx_v)

    def _scat2(j, carry):
        pltpu.sync_copy(val_v.at[j], acc2.at[idx_v.at[j]], add=True)
        return carry

    lax.fori_loop(0, R, _scat2, 0)

    plsc.subcore_barrier()

    @pl.when(s == 0)
    def _out():
        pltpu.sync_copy(acc1, out.at[c * 2])
        pltpu.sync_copy(acc2, out.at[c * 2 + 1])


_bincount = pl.kernel(
    _bincount_body,
    out_type=jax.ShapeDtypeStruct((4, N_PAD), jnp.float32),
    mesh=_mesh,
    scratch_types=[
        pltpu.VMEM((R, LANE), jnp.int32),
        pltpu.VMEM((R, LANE), jnp.float32),
        pltpu.VMEM_SHARED((N_PAD,), jnp.float32),
        pltpu.VMEM_SHARED((N_PAD,), jnp.float32),
    ],
)


def _edge_body(src1, dst1, src2, dst2, g0h, g1h, g2h, g3h, zeros_h, out,
               si_v, di_v, val_v, g0, g1, g2, g3, a0, a1, a2, a3):
    c = lax.axis_index("c")
    s = lax.axis_index("s")
    tid = c * NS + s

    @pl.when(s == 0)
    def _stage():
        pltpu.sync_copy(zeros_h, a0)
        pltpu.sync_copy(zeros_h, a1)
        pltpu.sync_copy(zeros_h, a2)
        pltpu.sync_copy(zeros_h, a3)
        pltpu.sync_copy(g0h, g0)
        pltpu.sync_copy(g1h, g1)
        pltpu.sync_copy(g2h, g2)
        pltpu.sync_copy(g3h, g3)

    plsc.subcore_barrier()

    def _pass(src, dst, glo, ghi, alo, ahi):
        pltpu.sync_copy(src.at[tid], si_v)
        pltpu.sync_copy(dst.at[tid], di_v)

        def _row(j, carry):
            pltpu.sync_copy(glo.at[si_v.at[j]], val_v.at[j])
            pltpu.sync_copy(val_v.at[j], alo.at[di_v.at[j]], add=True)
            pltpu.sync_copy(ghi.at[si_v.at[j]], val_v.at[j])
            pltpu.sync_copy(val_v.at[j], ahi.at[di_v.at[j]], add=True)
            return carry

        lax.fori_loop(0, R, _row, 0)

    _pass(src1, dst1, g1, g3, a1, a3)
    _pass(src2, dst2, g0, g2, a0, a2)

    plsc.subcore_barrier()

    @pl.when(s == 0)
    def _out():
        pltpu.sync_copy(a0, out.at[c * 4])
        pltpu.sync_copy(a1, out.at[c * 4 + 1])
        pltpu.sync_copy(a2, out.at[c * 4 + 2])
        pltpu.sync_copy(a3, out.at[c * 4 + 3])


_edge_scatter = pl.kernel(
    _edge_body,
    out_type=jax.ShapeDtypeStruct((8, N_PAD), jnp.float32),
    mesh=_mesh,
    scratch_types=[
        pltpu.VMEM((R, LANE), jnp.int32),
        pltpu.VMEM((R, LANE), jnp.int32),
        pltpu.VMEM((R, LANE), jnp.float32),
        pltpu.VMEM_SHARED((N_PAD,), jnp.float32),
        pltpu.VMEM_SHARED((N_PAD,), jnp.float32),
        pltpu.VMEM_SHARED((N_PAD,), jnp.float32),
        pltpu.VMEM_SHARED((N_PAD,), jnp.float32),
        pltpu.VMEM_SHARED((N_PAD,), jnp.float32),
        pltpu.VMEM_SHARED((N_PAD,), jnp.float32),
        pltpu.VMEM_SHARED((N_PAD,), jnp.float32),
        pltpu.VMEM_SHARED((N_PAD,), jnp.float32),
    ],
)


def _dense_body(cnt_ref, feat_ref, shuf_ref, w1_ref, b1_ref, w2_ref, b2_ref,
                bw_ref, bb_ref, g_ref, off_ref):
    cnt = cnt_ref[...]
    cnt1 = cnt[0:1] + cnt[2:3]
    cnt2 = cnt[1:2] + cnt[3:4]
    feat = feat_ref[...]
    shuf = shuf_ref[...]
    inv_n = jnp.float32(1.0 / N)
    m1 = jnp.dot(cnt1, feat, preferred_element_type=jnp.float32, precision=lax.Precision.HIGHEST) * inv_n
    m2 = jnp.dot(cnt2, feat, preferred_element_type=jnp.float32, precision=lax.Precision.HIGHEST) * inv_n
    b1 = b1_ref[...]
    b2 = b2_ref[...]
    c1 = jax.nn.sigmoid(jnp.dot(m1, w1_ref[...],
                                preferred_element_type=jnp.float32, precision=lax.Precision.HIGHEST) + b1)
    c2 = jax.nn.sigmoid(jnp.dot(m2, w2_ref[...],
                                preferred_element_type=jnp.float32, precision=lax.Precision.HIGHEST) + b2)
    dn = (((1,), (1,)), ((), ()))
    v1 = lax.dot_general(c1, bw_ref[...], dn,
                         preferred_element_type=jnp.float32, precision=lax.Precision.HIGHEST)
    v2 = lax.dot_general(c2, bw_ref[...], dn,
                         preferred_element_type=jnp.float32, precision=lax.Precision.HIGHEST)
    u_a = lax.dot_general(v1, w2_ref[...], dn,
                          preferred_element_type=jnp.float32, precision=lax.Precision.HIGHEST)
    u_b = lax.dot_general(v2, w1_ref[...], dn,
                          preferred_element_type=jnp.float32, precision=lax.Precision.HIGHEST)
    bb = jnp.sum(bb_ref[...])
    off_a = jnp.sum(b2 * v1) + bb
    off_b = jnp.sum(b1 * v2) + bb
    g_a = jnp.sum(feat * u_a, axis=1)
    g_b = jnp.sum(feat * u_b, axis=1)
    gs_a = jnp.sum(shuf * u_a, axis=1)
    gs_b = jnp.sum(shuf * u_b, axis=1)
    g_ref[...] = jnp.concatenate(
        [g_a[None, :], g_b[None, :], gs_a[None, :], gs_b[None, :]], axis=0)
    offs = jnp.concatenate(
        [jnp.full((1, 1), off_a, jnp.float32),
         jnp.full((1, 1), off_b, jnp.float32),
         jnp.full((1, 1), off_a, jnp.float32),
         jnp.full((1, 1), off_b, jnp.float32)], axis=0)
    off_ref[...] = jnp.broadcast_to(offs, (4, N))


def _combine_body(parts_ref, off_ref, out_ref):
    p = parts_ref[...]
    out_ref[...] = p[0:4] + p[4:8] + off_ref[...]


def kernel(feat, shuf_feat, edge_index, dif_edge_index, W1, b1, W2, b2, Bw, Bb):
    pad = E_PAD - E
    src1 = jnp.pad(edge_index[0], (0, pad), constant_values=N).reshape(NW, R, LANE)
    dst1 = jnp.pad(edge_index[1], (0, pad), constant_values=N).reshape(NW, R, LANE)
    src2 = jnp.pad(dif_edge_index[0], (0, pad), constant_values=N).reshape(NW, R, LANE)
    dst2 = jnp.pad(dif_edge_index[1], (0, pad), constant_values=N).reshape(NW, R, LANE)
    ones = jnp.ones((R, LANE), jnp.float32)
    zeros = jnp.zeros((N_PAD,), jnp.float32)

    cnt_parts = _bincount(src1, src2, ones, zeros)
    cnt = cnt_parts[:, :N]

    g4, off4 = pl.pallas_call(
        _dense_body,
        out_shape=(jax.ShapeDtypeStruct((4, N), jnp.float32),
                   jax.ShapeDtypeStruct((4, N), jnp.float32)),
    )(cnt, feat, shuf_feat, W1, b1.reshape(1, D), W2, b2.reshape(1, D),
      Bw, Bb.reshape(1, 1))

    gp = jnp.pad(g4, ((0, 0), (0, N_PAD - N)))
    parts = _edge_scatter(src1, dst1, src2, dst2,
                          gp[0], gp[1], gp[2], gp[3], zeros)

    out4 = pl.pallas_call(
        _combine_body,
        out_shape=jax.ShapeDtypeStruct((4, N), jnp.float32),
    )(parts[:, :N], off4)

    return out4.reshape(1, 4 * N)
